# Initial kernel scaffold; baseline (speedup 1.0000x reference)
import jax
import jax.numpy as jnp
from jax.experimental import pallas as pl


def kernel(nL, task_id, e_p):
    raise NotImplementedError("write your pallas kernel here")


# speedup vs baseline: 1.1557x; 1.1557x over previous
import functools

import jax
import jax.numpy as jnp
from jax import lax
from jax.experimental import pallas as pl
from jax.experimental.pallas import tpu as pltpu
from jax.experimental.pallas import tpu_sc as plsc

NUM_LAYERS = 12
N_TASKS = 1000
NUM_PROMPTS = 20
EMB_D = 128
BATCH = 1024
D = NUM_PROMPTS * EMB_D

NC = 2
NS = 16
NW = NC * NS
BPW = BATCH // NW
CHUNK = 16


def _sc_body(table_hbm, task_hbm, out_hbm, idx_all, idx_g, rows, sem):
    wid = lax.axis_index("s") * NC + lax.axis_index("c")
    base = wid * BPW
    pltpu.sync_copy(task_hbm.at[pl.ds(base, BPW)], idx_all)

    def layer_step(l, carry):
        for j in range(BPW // CHUNK):
            ids = idx_all[pl.ds(j * CHUNK, CHUNK)]
            idx_g[...] = ids + l * N_TASKS
            pltpu.async_copy(table_hbm.at[idx_g], rows, sem).wait()
            row_base = l * BATCH + base + j * CHUNK
            pltpu.sync_copy(rows, out_hbm.at[pl.ds(row_base, CHUNK)])
        return carry

    lax.fori_loop(0, NUM_LAYERS, layer_step, 0)


@functools.partial(
    pl.kernel,
    mesh=plsc.VectorSubcoreMesh(core_axis_name="c", subcore_axis_name="s"),
    out_type=jax.ShapeDtypeStruct((NUM_LAYERS * BATCH, D), jnp.float32),
    scratch_types=[
        pltpu.VMEM((BPW,), jnp.int32),
        pltpu.VMEM((CHUNK,), jnp.int32),
        pltpu.VMEM((CHUNK, D), jnp.float32),
        pltpu.SemaphoreType.DMA,
    ],
)
def _gather_sc(table_hbm, task_hbm, out_hbm, idx_all, idx_g, rows, sem):
    _sc_body(table_hbm, task_hbm, out_hbm, idx_all, idx_g, rows, sem)


def kernel(nL, task_id, e_p):
    table = e_p.reshape(NUM_LAYERS * N_TASKS, D)
    out = _gather_sc(table, task_id)
    return out.reshape(NUM_LAYERS, BATCH, NUM_PROMPTS, EMB_D)


# speedup vs baseline: 1.1995x; 1.0379x over previous
import functools

import jax
import jax.numpy as jnp
from jax import lax
from jax.experimental import pallas as pl
from jax.experimental.pallas import tpu as pltpu
from jax.experimental.pallas import tpu_sc as plsc

NUM_LAYERS = 12
N_TASKS = 1000
NUM_PROMPTS = 20
EMB_D = 128
BATCH = 1024
D = NUM_PROMPTS * EMB_D

NC = 2
NS = 16
NW = NC * NS
BPW = BATCH // NW
CHUNK = 16


def _sc_body(table, task, out, idx_all, idx_g0, idx_g1, rows0, rows1,
             sg0, sg1, sw0, sw1):
    wid = lax.axis_index("s") * NC + lax.axis_index("c")
    base = wid * BPW
    pltpu.sync_copy(task.at[pl.ds(base, BPW)], idx_all)

    idx_g = (idx_g0, idx_g1)
    rows = (rows0, rows1)
    sg = (sg0, sg1)
    sw = (sw0, sw1)

    def prep_and_fire(l, j, b):
        ids = idx_all[pl.ds(j * CHUNK, CHUNK)]
        idx_g[b][...] = ids + l * N_TASKS
        pltpu.async_copy(table.at[idx_g[b]], rows[b], sg[b])

    def wait_gather(b):
        pltpu.make_async_copy(table.at[idx_g[b]], rows[b], sg[b]).wait()

    def wait_write(b):
        pltpu.make_async_copy(rows[b], out.at[pl.ds(0, CHUNK)], sw[b]).wait()

    prep_and_fire(0, 0, 0)

    def layer_step(l, carry):
        @pl.when(l > 0)
        def _():
            wait_write(1)
        prep_and_fire(l, 1, 1)
        wait_gather(0)
        pltpu.async_copy(rows[0], out.at[pl.ds(l * BATCH + base, CHUNK)], sw[0])

        @pl.when(l < NUM_LAYERS - 1)
        def _():
            wait_write(0)
            prep_and_fire(l + 1, 0, 0)
        wait_gather(1)
        pltpu.async_copy(
            rows[1], out.at[pl.ds(l * BATCH + base + CHUNK, CHUNK)], sw[1])
        return carry

    lax.fori_loop(0, NUM_LAYERS, layer_step, 0)

    wait_write(0)
    wait_write(1)


@functools.partial(
    pl.kernel,
    mesh=plsc.VectorSubcoreMesh(core_axis_name="c", subcore_axis_name="s"),
    out_type=jax.ShapeDtypeStruct((NUM_LAYERS * BATCH, D), jnp.float32),
    scratch_types=[
        pltpu.VMEM((BPW,), jnp.int32),
        pltpu.VMEM((CHUNK,), jnp.int32),
        pltpu.VMEM((CHUNK,), jnp.int32),
        pltpu.VMEM((CHUNK, D), jnp.float32),
        pltpu.VMEM((CHUNK, D), jnp.float32),
        pltpu.SemaphoreType.DMA,
        pltpu.SemaphoreType.DMA,
        pltpu.SemaphoreType.DMA,
        pltpu.SemaphoreType.DMA,
    ],
)
def _gather_sc(table, task, out, idx_all, idx_g0, idx_g1, rows0, rows1,
               sg0, sg1, sw0, sw1):
    _sc_body(table, task, out, idx_all, idx_g0, idx_g1, rows0, rows1,
             sg0, sg1, sw0, sw1)


def kernel(nL, task_id, e_p):
    table = e_p.reshape(NUM_LAYERS * N_TASKS, D)
    out = _gather_sc(table, task_id)
    return out.reshape(NUM_LAYERS, BATCH, NUM_PROMPTS, EMB_D)


# speedup vs baseline: 2.1203x; 1.7677x over previous
import functools

import jax
import jax.numpy as jnp
from jax import lax
from jax.experimental import pallas as pl
from jax.experimental.pallas import tpu as pltpu
from jax.experimental.pallas import tpu_sc as plsc

NUM_LAYERS = 12
N_TASKS = 1000
NUM_PROMPTS = 20
EMB_D = 128
BATCH = 1024
D = NUM_PROMPTS * EMB_D

NC = 2
NS = 16
NW = NC * NS
BPW = BATCH // NW
CHUNK = 16


def _sc_body(table, task, out, idx_all, idx_g0, idx_g1, rows0, rows1,
             sg0, sg1, sw0, sw1):
    wid = lax.axis_index("s") * NC + lax.axis_index("c")
    base = wid * BPW
    pltpu.sync_copy(task.at[pl.ds(base, BPW)], idx_all)

    idx_g = (idx_g0, idx_g1)
    rows = (rows0, rows1)
    sg = (sg0, sg1)
    sw = (sw0, sw1)

    def prep_and_fire(l, j, b):
        ids = idx_all[pl.ds(j * CHUNK, CHUNK)]
        idx_g[b][...] = ids + l * N_TASKS
        pltpu.async_copy(table.at[idx_g[b]], rows[b], sg[b])

    def wait_gather(b):
        pltpu.make_async_copy(table.at[idx_g[b]], rows[b], sg[b]).wait()

    def wait_write(b):
        pltpu.make_async_copy(rows[b], out.at[pl.ds(0, CHUNK)], sw[b]).wait()

    prep_and_fire(0, 0, 0)

    def layer_step(l, carry):
        @pl.when(l > 0)
        def _():
            wait_write(1)
        prep_and_fire(l, 1, 1)
        wait_gather(0)
        pltpu.async_copy(rows[0], out.at[pl.ds(l * BATCH + base, CHUNK)], sw[0])

        @pl.when(l < NUM_LAYERS - 1)
        def _():
            wait_write(0)
            prep_and_fire(l + 1, 0, 0)
        wait_gather(1)
        pltpu.async_copy(
            rows[1], out.at[pl.ds(l * BATCH + base + CHUNK, CHUNK)], sw[1])
        return carry

    lax.fori_loop(0, NUM_LAYERS, layer_step, 0)

    wait_write(0)
    wait_write(1)


@functools.partial(
    pl.kernel,
    mesh=plsc.VectorSubcoreMesh(core_axis_name="c", subcore_axis_name="s"),
    out_type=jax.ShapeDtypeStruct((NUM_LAYERS * BATCH, NUM_PROMPTS, EMB_D), jnp.float32),
    scratch_types=[
        pltpu.VMEM((BPW,), jnp.int32),
        pltpu.VMEM((CHUNK,), jnp.int32),
        pltpu.VMEM((CHUNK,), jnp.int32),
        pltpu.VMEM((CHUNK, NUM_PROMPTS, EMB_D), jnp.float32),
        pltpu.VMEM((CHUNK, NUM_PROMPTS, EMB_D), jnp.float32),
        pltpu.SemaphoreType.DMA,
        pltpu.SemaphoreType.DMA,
        pltpu.SemaphoreType.DMA,
        pltpu.SemaphoreType.DMA,
    ],
)
def _gather_sc(table, task, out, idx_all, idx_g0, idx_g1, rows0, rows1,
               sg0, sg1, sw0, sw1):
    _sc_body(table, task, out, idx_all, idx_g0, idx_g1, rows0, rows1,
             sg0, sg1, sw0, sw1)


def kernel(nL, task_id, e_p):
    table = e_p.reshape(NUM_LAYERS * N_TASKS, NUM_PROMPTS, EMB_D)
    out = _gather_sc(table, task_id)
    return out.reshape(NUM_LAYERS, BATCH, NUM_PROMPTS, EMB_D)
